# baseline (speedup 1.0000x reference)
import jax
import jax.numpy as jnp
from jax import lax
from jax.experimental import pallas as pl
from jax.experimental.pallas import tpu as pltpu
from jax.experimental.pallas import tpu_sc as plsc

_NS = 16
_L = 16
_D = 512


def _body(l2_hbm, l1_hbm, l0_hbm, out_hbm, buf, sem):
    w = lax.axis_index("s")
    iota = lax.iota(jnp.int32, _L)

    @pl.when(w < 8)
    def _():
        j = w * _L + iota
        idx = (j >> 1) + (j & 1) * 200
        gather = pltpu.async_copy(l0_hbm.at[idx], buf, sem)
        base = w * _L
        pltpu.sync_copy(l2_hbm.at[pl.ds(base, _L)], out_hbm.at[pl.ds(base, _L)])
        gather.wait()
        pltpu.sync_copy(buf, out_hbm.at[pl.ds(256 + base, _L)])

    @pl.when(w >= 8)
    def _():
        g = w - 8
        idx = (g * _L + iota) * 3
        pltpu.async_copy(l1_hbm.at[idx], buf, sem).wait()
        pltpu.sync_copy(buf, out_hbm.at[pl.ds(128 + g * _L, _L)])


def kernel(layer_2, layer_1, layer_0):
    mesh = plsc.VectorSubcoreMesh(
        core_axis_name="c", subcore_axis_name="s",
        num_cores=1, num_subcores=_NS,
    )
    f = pl.kernel(
        _body,
        out_type=jax.ShapeDtypeStruct((384, _D), jnp.float32),
        mesh=mesh,
        scratch_types=[
            pltpu.VMEM((_L, _D), jnp.float32),
            pltpu.SemaphoreType.DMA,
        ],
    )
    return f(layer_2, layer_1, layer_0)
